# Initial kernel scaffold; baseline (speedup 1.0000x reference)
import jax
import jax.numpy as jnp
from jax.experimental import pallas as pl


def kernel(X, y):
    raise NotImplementedError("write your pallas kernel here")


# speedup vs baseline: 17.8586x; 17.8586x over previous
import jax
import jax.numpy as jnp
from jax import lax
from jax.experimental import pallas as pl
from jax.experimental.pallas import tpu as pltpu
from jax.scipy.special import digamma as _digamma

N = 4096
D = 256
K = 5
N_CLASSES = 10
BR = 256
NB = N // BR
HIGH_CST = 10000000.0
TBL = 4224
_EXACT = lax.Precision.HIGHEST


def _mi_kernel(xall_ref, xr_ref, sqc_ref, sqr_ref, yc_ref, yr_ref, tbl_ref,
               base_ref, out_ref, acc_ref):
    i = pl.program_id(0)

    xall = xall_ref[...]
    xr = xr_ref[...]
    sqc = sqc_ref[...]
    sqr = sqr_ref[...]
    yc = yc_ref[...]
    yr = yr_ref[...]
    tbl = tbl_ref[...]

    p = lax.dot_general(xr, xall, (((1,), (1,)), ((), ())),
                        preferred_element_type=jnp.float32)
    d2 = (sqr + sqc) - 2.0 * p
    dm = jnp.sqrt(jnp.maximum(d2, 0.0))

    a = jnp.where(yr == yc, dm, HIGH_CST)

    remaining = jnp.full((BR, 1), K + 1, jnp.int32)
    anchor = jnp.full((BR, 1), HIGH_CST, jnp.float32)
    for _ in range(K + 1):
        mn = jnp.min(a, axis=1, keepdims=True)
        anchor = jnp.where(remaining > 0, mn, anchor)
        eq = a == mn
        cnt = jnp.sum(eq.astype(jnp.int32), axis=1, keepdims=True)
        remaining = remaining - cnt
        a = jnp.where(eq, HIGH_CST, a)

    m = jnp.sum((dm <= anchor).astype(jnp.int32), axis=1,
                keepdims=True) - 1
    oh = (lax.broadcasted_iota(jnp.int32, (BR, TBL), 1) == m)
    psim = lax.dot_general(oh.astype(jnp.float32), tbl,
                           (((1,), (0,)), ((), ())),
                           precision=_EXACT,
                           preferred_element_type=jnp.float32)
    dig_sum = jnp.sum(psim, keepdims=True)

    @pl.when(i == 0)
    def _():
        acc_ref[...] = jnp.zeros_like(acc_ref)

    acc_ref[...] = acc_ref[...] + dig_sum

    @pl.when(i == NB - 1)
    def _():
        cls = lax.broadcasted_iota(jnp.int32, (N_CLASSES, 1), 0)
        counts = jnp.sum((cls == yc).astype(jnp.int32), axis=1,
                         keepdims=True)
        ohc = (lax.broadcasted_iota(jnp.int32, (N_CLASSES, TBL), 1) == counts)
        psin = lax.dot_general(ohc.astype(jnp.float32), tbl,
                               (((1,), (0,)), ((), ())),
                               precision=_EXACT,
                               preferred_element_type=jnp.float32)
        avg_nx = jnp.sum((counts.astype(jnp.float32) * (1.0 / N)) * psin,
                         keepdims=True)
        mi = base_ref[...] - avg_nx - acc_ref[...] * (1.0 / N)
        out_ref[...] = jnp.maximum(mi / jnp.log(2.0), 0.0)


def _mi_call(x, sq_row, sq_col, y_col, y_row, tbl, base):
    return pl.pallas_call(
        _mi_kernel,
        grid=(NB,),
        in_specs=[
            pl.BlockSpec((N, D), lambda i: (0, 0)),
            pl.BlockSpec((BR, D), lambda i: (i, 0)),
            pl.BlockSpec((1, N), lambda i: (0, 0)),
            pl.BlockSpec((BR, 1), lambda i: (i, 0)),
            pl.BlockSpec((1, N), lambda i: (0, 0)),
            pl.BlockSpec((BR, 1), lambda i: (i, 0)),
            pl.BlockSpec((TBL, 1), lambda i: (0, 0)),
            pl.BlockSpec((1, 1), lambda i: (0, 0)),
        ],
        out_specs=pl.BlockSpec((1, 1), lambda i: (0, 0)),
        out_shape=jax.ShapeDtypeStruct((1, 1), jnp.float32),
        scratch_shapes=[pltpu.VMEM((1, 1), jnp.float32)],
    )(x, x, sq_row, sq_col, y_col, y_row, tbl, base)


def kernel(X, y):
    y32 = y.astype(jnp.int32)
    sq = jnp.sum(X * X, axis=1)
    tbl = _digamma(jnp.arange(TBL, dtype=jnp.float32)).reshape(TBL, 1)
    tbl = jnp.where(jnp.isfinite(tbl), tbl, 0.0)
    base = (_digamma(jnp.float32(N)) + _digamma(jnp.float32(K))).reshape(1, 1)
    out = _mi_call(X, sq.reshape(1, N), sq.reshape(N, 1),
                   y32.reshape(1, N), y32.reshape(N, 1), tbl, base)
    return out.reshape(())


# speedup vs baseline: 28.0289x; 1.5695x over previous
import jax
import jax.numpy as jnp
from jax import lax
from jax.experimental import pallas as pl
from jax.experimental.pallas import tpu as pltpu
from jax.scipy.special import digamma as _digamma

N = 4096
D = 256
K = 5
N_CLASSES = 10
BR = 256
NB = N // BR
HIGH_CST = 10000000.0
TBL_HI = 40
TBL_LO = 128
_EXACT = lax.Precision.HIGHEST


def _tbl_gather(idx, tbl2d, rows):
    hi = lax.shift_right_logical(idx, 7)
    lo = lax.bitwise_and(idx, 127)
    oh_hi = (lax.broadcasted_iota(jnp.int32, (rows, TBL_HI), 1) == hi)
    row = lax.dot_general(oh_hi.astype(jnp.float32), tbl2d,
                          (((1,), (0,)), ((), ())),
                          precision=_EXACT,
                          preferred_element_type=jnp.float32)
    oh_lo = (lax.broadcasted_iota(jnp.int32, (rows, TBL_LO), 1) == lo)
    return jnp.sum(jnp.where(oh_lo, row, 0.0), axis=1, keepdims=True)


def _mi_kernel(xall_ref, xr_ref, sqc_ref, sqr_ref, yc_ref, yr_ref, tbl_ref,
               base_ref, out_ref, acc_ref):
    i = pl.program_id(0)

    xall = xall_ref[...]
    xr = xr_ref[...]
    sqc = sqc_ref[...]
    sqr = sqr_ref[...]
    yc = yc_ref[...]
    yr = yr_ref[...]
    tbl = tbl_ref[...]

    p = lax.dot_general(xr, xall, (((1,), (1,)), ((), ())),
                        preferred_element_type=jnp.float32)
    d2 = (sqr + sqc) - 2.0 * p
    dm = jnp.sqrt(jnp.maximum(d2, 0.0))

    a = jnp.where(yr == yc, dm, HIGH_CST)

    remaining = jnp.full((BR, 1), K + 1, jnp.int32)
    anchor = jnp.full((BR, 1), HIGH_CST, jnp.float32)
    for _ in range(K + 1):
        mn = jnp.min(a, axis=1, keepdims=True)
        anchor = jnp.where(remaining > 0, mn, anchor)
        eq = a == mn
        cnt = jnp.sum(eq.astype(jnp.int32), axis=1, keepdims=True)
        remaining = remaining - cnt
        a = jnp.where(eq, HIGH_CST, a)

    m = jnp.sum((dm <= anchor).astype(jnp.int32), axis=1,
                keepdims=True) - 1
    psim = _tbl_gather(m, tbl, BR)
    dig_sum = jnp.sum(psim, keepdims=True)

    @pl.when(i == 0)
    def _():
        acc_ref[...] = jnp.zeros_like(acc_ref)

    acc_ref[...] = acc_ref[...] + dig_sum

    @pl.when(i == NB - 1)
    def _():
        cls = lax.broadcasted_iota(jnp.int32, (N_CLASSES, 1), 0)
        counts = jnp.sum((cls == yc).astype(jnp.int32), axis=1,
                         keepdims=True)
        psin = _tbl_gather(counts, tbl, N_CLASSES)
        avg_nx = jnp.sum((counts.astype(jnp.float32) * (1.0 / N)) * psin,
                         keepdims=True)
        mi = base_ref[...] - avg_nx - acc_ref[...] * (1.0 / N)
        out_ref[...] = jnp.maximum(mi / jnp.log(2.0), 0.0)


def _mi_call(x, sq_row, sq_col, y_col, y_row, tbl, base):
    return pl.pallas_call(
        _mi_kernel,
        grid=(NB,),
        in_specs=[
            pl.BlockSpec((N, D), lambda i: (0, 0)),
            pl.BlockSpec((BR, D), lambda i: (i, 0)),
            pl.BlockSpec((1, N), lambda i: (0, 0)),
            pl.BlockSpec((BR, 1), lambda i: (i, 0)),
            pl.BlockSpec((1, N), lambda i: (0, 0)),
            pl.BlockSpec((BR, 1), lambda i: (i, 0)),
            pl.BlockSpec((TBL_HI, TBL_LO), lambda i: (0, 0)),
            pl.BlockSpec((1, 1), lambda i: (0, 0)),
        ],
        out_specs=pl.BlockSpec((1, 1), lambda i: (0, 0)),
        out_shape=jax.ShapeDtypeStruct((1, 1), jnp.float32),
        scratch_shapes=[pltpu.VMEM((1, 1), jnp.float32)],
    )(x, x, sq_row, sq_col, y_col, y_row, tbl, base)


def kernel(X, y):
    y32 = y.astype(jnp.int32)
    sq = jnp.sum(X * X, axis=1)
    tbl = _digamma(jnp.arange(TBL_HI * TBL_LO, dtype=jnp.float32))
    tbl = jnp.where(jnp.isfinite(tbl), tbl, 0.0)
    tbl = tbl.reshape(TBL_HI, TBL_LO)
    base = (_digamma(jnp.float32(N)) + _digamma(jnp.float32(K))).reshape(1, 1)
    out = _mi_call(X, sq.reshape(1, N), sq.reshape(N, 1),
                   y32.reshape(1, N), y32.reshape(N, 1), tbl, base)
    return out.reshape(())


# speedup vs baseline: 36.1558x; 1.2899x over previous
import jax
import jax.numpy as jnp
from jax import lax
from jax.experimental import pallas as pl
from jax.experimental.pallas import tpu as pltpu
from jax.scipy.special import digamma as _digamma

N = 4096
D = 256
K = 5
N_CLASSES = 10
BR = 256
NB = N // BR
HIGH_CST = 10000000.0
TBL_HI = 40
TBL_LO = 128
_EXACT = lax.Precision.HIGHEST


def _tbl_gather(idx, tbl2d, rows):
    hi = lax.shift_right_logical(idx, 7)
    lo = lax.bitwise_and(idx, 127)
    oh_hi = (lax.broadcasted_iota(jnp.int32, (rows, TBL_HI), 1) == hi)
    row = lax.dot_general(oh_hi.astype(jnp.float32), tbl2d,
                          (((1,), (0,)), ((), ())),
                          precision=_EXACT,
                          preferred_element_type=jnp.float32)
    oh_lo = (lax.broadcasted_iota(jnp.int32, (rows, TBL_LO), 1) == lo)
    return jnp.sum(jnp.where(oh_lo, row, 0.0), axis=1, keepdims=True)


def _mi_kernel(xall_ref, xr_ref, sqc_ref, sqr_ref, yc_ref, yr_ref, tbl_ref,
               base_ref, out_ref, acc_ref):
    i = pl.program_id(0)

    xall = xall_ref[...]
    xr = xr_ref[...]
    sqc = sqc_ref[...]
    sqr = sqr_ref[...]
    yc = yc_ref[...]
    yr = yr_ref[...]
    tbl = tbl_ref[...]

    p = lax.dot_general(xr, xall, (((1,), (1,)), ((), ())),
                        preferred_element_type=jnp.float32)
    d2 = (sqr + sqc) - 2.0 * p
    dm = jnp.sqrt(jnp.maximum(d2, 0.0))

    a = jnp.where(yr == yc, dm, HIGH_CST)

    rs = [jnp.full((BR, 128), HIGH_CST, jnp.float32) for _ in range(K + 1)]
    for t in range(N // 128):
        v = a[:, t * 128:(t + 1) * 128]
        for j in range(K + 1):
            lo = jnp.minimum(rs[j], v)
            v = jnp.maximum(rs[j], v)
            rs[j] = lo
    cand = jnp.concatenate(rs, axis=1)

    remaining = jnp.full((BR, 1), float(K + 1), jnp.float32)
    anchor = jnp.full((BR, 1), HIGH_CST, jnp.float32)
    for _ in range(K + 1):
        mn = jnp.min(cand, axis=1, keepdims=True)
        anchor = jnp.where(remaining > 0.0, mn, anchor)
        eq = cand == mn
        cnt = jnp.sum(jnp.where(eq, 1.0, 0.0), axis=1, keepdims=True)
        remaining = remaining - cnt
        cand = jnp.where(eq, HIGH_CST, cand)

    m = jnp.sum((dm <= anchor).astype(jnp.int32), axis=1,
                keepdims=True) - 1
    psim = _tbl_gather(m, tbl, BR)
    dig_sum = jnp.sum(psim, keepdims=True)

    @pl.when(i == 0)
    def _():
        acc_ref[...] = jnp.zeros_like(acc_ref)

    acc_ref[...] = acc_ref[...] + dig_sum

    @pl.when(i == NB - 1)
    def _():
        cls = lax.broadcasted_iota(jnp.int32, (N_CLASSES, 1), 0)
        counts = jnp.sum((cls == yc).astype(jnp.int32), axis=1,
                         keepdims=True)
        psin = _tbl_gather(counts, tbl, N_CLASSES)
        avg_nx = jnp.sum((counts.astype(jnp.float32) * (1.0 / N)) * psin,
                         keepdims=True)
        mi = base_ref[...] - avg_nx - acc_ref[...] * (1.0 / N)
        out_ref[...] = jnp.maximum(mi / jnp.log(2.0), 0.0)


def _mi_call(x, sq_row, sq_col, y_col, y_row, tbl, base):
    return pl.pallas_call(
        _mi_kernel,
        grid=(NB,),
        in_specs=[
            pl.BlockSpec((N, D), lambda i: (0, 0)),
            pl.BlockSpec((BR, D), lambda i: (i, 0)),
            pl.BlockSpec((1, N), lambda i: (0, 0)),
            pl.BlockSpec((BR, 1), lambda i: (i, 0)),
            pl.BlockSpec((1, N), lambda i: (0, 0)),
            pl.BlockSpec((BR, 1), lambda i: (i, 0)),
            pl.BlockSpec((TBL_HI, TBL_LO), lambda i: (0, 0)),
            pl.BlockSpec((1, 1), lambda i: (0, 0)),
        ],
        out_specs=pl.BlockSpec((1, 1), lambda i: (0, 0)),
        out_shape=jax.ShapeDtypeStruct((1, 1), jnp.float32),
        scratch_shapes=[pltpu.VMEM((1, 1), jnp.float32)],
    )(x, x, sq_row, sq_col, y_col, y_row, tbl, base)


def kernel(X, y):
    y32 = y.astype(jnp.int32)
    sq = jnp.sum(X * X, axis=1)
    tbl = _digamma(jnp.arange(TBL_HI * TBL_LO, dtype=jnp.float32))
    tbl = jnp.where(jnp.isfinite(tbl), tbl, 0.0)
    tbl = tbl.reshape(TBL_HI, TBL_LO)
    base = (_digamma(jnp.float32(N)) + _digamma(jnp.float32(K))).reshape(1, 1)
    out = _mi_call(X, sq.reshape(1, N), sq.reshape(N, 1),
                   y32.reshape(1, N), y32.reshape(N, 1), tbl, base)
    return out.reshape(())


# speedup vs baseline: 37.0493x; 1.0247x over previous
import jax
import jax.numpy as jnp
from jax import lax
from jax.experimental import pallas as pl
from jax.experimental.pallas import tpu as pltpu
from jax.scipy.special import digamma as _digamma

N = 4096
D = 256
K = 5
N_CLASSES = 10
BR = 256
NB = N // BR
HIGH_CST = 10000000.0
TBL_HI = 40
TBL_LO = 128
_EXACT = lax.Precision.HIGHEST


def _tbl_gather(idx, tbl2d, rows):
    hi = lax.shift_right_logical(idx, 7)
    lo = lax.bitwise_and(idx, 127)
    oh_hi = (lax.broadcasted_iota(jnp.int32, (rows, TBL_HI), 1) == hi)
    row = lax.dot_general(oh_hi.astype(jnp.float32), tbl2d,
                          (((1,), (0,)), ((), ())),
                          precision=_EXACT,
                          preferred_element_type=jnp.float32)
    oh_lo = (lax.broadcasted_iota(jnp.int32, (rows, TBL_LO), 1) == lo)
    return jnp.sum(jnp.where(oh_lo, row, 0.0), axis=1, keepdims=True)


def _mi_kernel(xall_ref, sqc_ref, sqr_ref, yc_ref, yr_ref, tbl_ref,
               base_ref, out_ref, acc_ref):
    i = pl.program_id(0)

    xall = xall_ref[...]
    xr = xall_ref[pl.ds(i * BR, BR), :]
    sqc = sqc_ref[...]
    sqr = sqr_ref[...]
    yc = yc_ref[...]
    yr = yr_ref[...]
    tbl = tbl_ref[...]

    p = lax.dot_general(xr, xall, (((1,), (1,)), ((), ())),
                        preferred_element_type=jnp.float32)
    d2 = (sqr + sqc) - 2.0 * p
    dm = jnp.sqrt(jnp.maximum(d2, 0.0))

    a = jnp.where(yr == yc, dm, HIGH_CST)

    rs = [jnp.full((BR, 128), HIGH_CST, jnp.float32) for _ in range(K + 1)]
    for t in range(N // 128):
        v = a[:, t * 128:(t + 1) * 128]
        for j in range(K + 1):
            lo = jnp.minimum(rs[j], v)
            v = jnp.maximum(rs[j], v)
            rs[j] = lo
    cand = jnp.concatenate(rs, axis=1)

    remaining = jnp.full((BR, 1), float(K + 1), jnp.float32)
    anchor = jnp.full((BR, 1), HIGH_CST, jnp.float32)
    for _ in range(K + 1):
        mn = jnp.min(cand, axis=1, keepdims=True)
        anchor = jnp.where(remaining > 0.0, mn, anchor)
        eq = cand == mn
        cnt = jnp.sum(jnp.where(eq, 1.0, 0.0), axis=1, keepdims=True)
        remaining = remaining - cnt
        cand = jnp.where(eq, HIGH_CST, cand)

    m = jnp.sum((dm <= anchor).astype(jnp.int32), axis=1,
                keepdims=True) - 1
    psim = _tbl_gather(m, tbl, BR)
    dig_sum = jnp.sum(psim, keepdims=True)

    @pl.when(i == 0)
    def _():
        acc_ref[...] = jnp.zeros_like(acc_ref)

    acc_ref[...] = acc_ref[...] + dig_sum

    @pl.when(i == NB - 1)
    def _():
        cls = lax.broadcasted_iota(jnp.int32, (N_CLASSES, 1), 0)
        counts = jnp.sum((cls == yc).astype(jnp.int32), axis=1,
                         keepdims=True)
        psin = _tbl_gather(counts, tbl, N_CLASSES)
        avg_nx = jnp.sum((counts.astype(jnp.float32) * (1.0 / N)) * psin,
                         keepdims=True)
        mi = base_ref[...] - avg_nx - acc_ref[...] * (1.0 / N)
        out_ref[...] = jnp.maximum(mi / jnp.log(2.0), 0.0)


def _mi_call(x, sq_row, sq_col, y_col, y_row, tbl, base):
    return pl.pallas_call(
        _mi_kernel,
        grid=(NB,),
        in_specs=[
            pl.BlockSpec((N, D), lambda i: (0, 0)),
            pl.BlockSpec((1, N), lambda i: (0, 0)),
            pl.BlockSpec((BR, 1), lambda i: (i, 0)),
            pl.BlockSpec((1, N), lambda i: (0, 0)),
            pl.BlockSpec((BR, 1), lambda i: (i, 0)),
            pl.BlockSpec((TBL_HI, TBL_LO), lambda i: (0, 0)),
            pl.BlockSpec((1, 1), lambda i: (0, 0)),
        ],
        out_specs=pl.BlockSpec((1, 1), lambda i: (0, 0)),
        out_shape=jax.ShapeDtypeStruct((1, 1), jnp.float32),
        scratch_shapes=[pltpu.VMEM((1, 1), jnp.float32)],
    )(x, sq_row, sq_col, y_col, y_row, tbl, base)


def kernel(X, y):
    y32 = y.astype(jnp.int32)
    sq = jnp.sum(X * X, axis=1)
    tbl = _digamma(jnp.arange(TBL_HI * TBL_LO, dtype=jnp.float32))
    tbl = jnp.where(jnp.isfinite(tbl), tbl, 0.0)
    tbl = tbl.reshape(TBL_HI, TBL_LO)
    base = (_digamma(jnp.float32(N)) + _digamma(jnp.float32(K))).reshape(1, 1)
    out = _mi_call(X, sq.reshape(1, N), sq.reshape(N, 1),
                   y32.reshape(1, N), y32.reshape(N, 1), tbl, base)
    return out.reshape(())


# speedup vs baseline: 38.2238x; 1.0317x over previous
import jax
import jax.numpy as jnp
from jax import lax
from jax.experimental import pallas as pl
from jax.experimental.pallas import tpu as pltpu
from jax.scipy.special import digamma as _digamma

N = 4096
D = 256
K = 5
N_CLASSES = 10
BR = 512
NB = N // BR
HIGH_CST = 10000000.0
TBL_HI = 40
TBL_LO = 128
_EXACT = lax.Precision.HIGHEST


def _tbl_gather(idx, tbl2d, rows):
    hi = lax.shift_right_logical(idx, 7)
    lo = lax.bitwise_and(idx, 127)
    oh_hi = (lax.broadcasted_iota(jnp.int32, (rows, TBL_HI), 1) == hi)
    row = lax.dot_general(oh_hi.astype(jnp.float32), tbl2d,
                          (((1,), (0,)), ((), ())),
                          precision=_EXACT,
                          preferred_element_type=jnp.float32)
    oh_lo = (lax.broadcasted_iota(jnp.int32, (rows, TBL_LO), 1) == lo)
    return jnp.sum(jnp.where(oh_lo, row, 0.0), axis=1, keepdims=True)


def _mi_kernel(xall_ref, sqc_ref, sqr_ref, yc_ref, yr_ref, tbl_ref,
               base_ref, out_ref, acc_ref):
    i = pl.program_id(0)

    xall = xall_ref[...]
    xr = xall_ref[pl.ds(i * BR, BR), :]
    sqc = sqc_ref[...]
    sqr = sqr_ref[...]
    yc = yc_ref[...]
    yr = yr_ref[...]
    tbl = tbl_ref[...]

    p = lax.dot_general(xr, xall, (((1,), (1,)), ((), ())),
                        preferred_element_type=jnp.float32)
    d2 = (sqr + sqc) - 2.0 * p
    dm = jnp.sqrt(jnp.maximum(d2, 0.0))

    a = jnp.where(yr == yc, dm, HIGH_CST)

    rs = [jnp.full((BR, 128), HIGH_CST, jnp.float32) for _ in range(K + 1)]
    for t in range(N // 128):
        v = a[:, t * 128:(t + 1) * 128]
        for j in range(K + 1):
            lo = jnp.minimum(rs[j], v)
            v = jnp.maximum(rs[j], v)
            rs[j] = lo
    cand = jnp.concatenate(rs, axis=1)

    remaining = jnp.full((BR, 1), float(K + 1), jnp.float32)
    anchor = jnp.full((BR, 1), HIGH_CST, jnp.float32)
    for _ in range(K + 1):
        mn = jnp.min(cand, axis=1, keepdims=True)
        anchor = jnp.where(remaining > 0.0, mn, anchor)
        eq = cand == mn
        cnt = jnp.sum(jnp.where(eq, 1.0, 0.0), axis=1, keepdims=True)
        remaining = remaining - cnt
        cand = jnp.where(eq, HIGH_CST, cand)

    m = jnp.sum((dm <= anchor).astype(jnp.int32), axis=1,
                keepdims=True) - 1
    psim = _tbl_gather(m, tbl, BR)
    dig_sum = jnp.sum(psim, keepdims=True)

    @pl.when(i == 0)
    def _():
        acc_ref[...] = jnp.zeros_like(acc_ref)

    acc_ref[...] = acc_ref[...] + dig_sum

    @pl.when(i == NB - 1)
    def _():
        cls = lax.broadcasted_iota(jnp.int32, (N_CLASSES, 1), 0)
        counts = jnp.sum((cls == yc).astype(jnp.int32), axis=1,
                         keepdims=True)
        psin = _tbl_gather(counts, tbl, N_CLASSES)
        avg_nx = jnp.sum((counts.astype(jnp.float32) * (1.0 / N)) * psin,
                         keepdims=True)
        mi = base_ref[...] - avg_nx - acc_ref[...] * (1.0 / N)
        out_ref[...] = jnp.maximum(mi / jnp.log(2.0), 0.0)


def _mi_call(x, sq_row, sq_col, y_col, y_row, tbl, base):
    return pl.pallas_call(
        _mi_kernel,
        grid=(NB,),
        in_specs=[
            pl.BlockSpec((N, D), lambda i: (0, 0)),
            pl.BlockSpec((1, N), lambda i: (0, 0)),
            pl.BlockSpec((BR, 1), lambda i: (i, 0)),
            pl.BlockSpec((1, N), lambda i: (0, 0)),
            pl.BlockSpec((BR, 1), lambda i: (i, 0)),
            pl.BlockSpec((TBL_HI, TBL_LO), lambda i: (0, 0)),
            pl.BlockSpec((1, 1), lambda i: (0, 0)),
        ],
        out_specs=pl.BlockSpec((1, 1), lambda i: (0, 0)),
        out_shape=jax.ShapeDtypeStruct((1, 1), jnp.float32),
        scratch_shapes=[pltpu.VMEM((1, 1), jnp.float32)],
    )(x, sq_row, sq_col, y_col, y_row, tbl, base)


def kernel(X, y):
    y32 = y.astype(jnp.int32)
    sq = jnp.sum(X * X, axis=1)
    tbl = _digamma(jnp.arange(TBL_HI * TBL_LO, dtype=jnp.float32))
    tbl = jnp.where(jnp.isfinite(tbl), tbl, 0.0)
    tbl = tbl.reshape(TBL_HI, TBL_LO)
    base = (_digamma(jnp.float32(N)) + _digamma(jnp.float32(K))).reshape(1, 1)
    out = _mi_call(X, sq.reshape(1, N), sq.reshape(N, 1),
                   y32.reshape(1, N), y32.reshape(N, 1), tbl, base)
    return out.reshape(())


# speedup vs baseline: 38.3960x; 1.0045x over previous
import jax
import jax.numpy as jnp
from jax import lax
from jax.experimental import pallas as pl
from jax.experimental.pallas import tpu as pltpu
from jax.scipy.special import digamma as _digamma

N = 4096
D = 256
K = 5
N_CLASSES = 10
BR = 512
NB = N // BR
HIGH_CST = 10000000.0
TBL_HI = 40
TBL_LO = 128
_EXACT = lax.Precision.HIGHEST


def _tbl_gather(idx, tbl2d, rows):
    hi = lax.shift_right_logical(idx, 7)
    lo = lax.bitwise_and(idx, 127)
    oh_hi = (lax.broadcasted_iota(jnp.int32, (rows, TBL_HI), 1) == hi)
    row = lax.dot_general(oh_hi.astype(jnp.float32), tbl2d,
                          (((1,), (0,)), ((), ())),
                          precision=_EXACT,
                          preferred_element_type=jnp.float32)
    oh_lo = (lax.broadcasted_iota(jnp.int32, (rows, TBL_LO), 1) == lo)
    return jnp.sum(jnp.where(oh_lo, row, 0.0), axis=1, keepdims=True)


def _mi_kernel(xall_ref, sqc_ref, sqr_ref, yc_ref, yr_ref, tbl_ref,
               base_ref, out_ref, acc_ref):
    i = pl.program_id(0)

    xall = xall_ref[...]
    xr = xall_ref[pl.ds(i * BR, BR), :]
    sqc = sqc_ref[...]
    sqr = sqr_ref[...]
    yc = yc_ref[...]
    yr = yr_ref[...]
    tbl = tbl_ref[...]

    p = lax.dot_general(xr, xall, (((1,), (1,)), ((), ())),
                        preferred_element_type=jnp.float32)
    d2 = (sqr + sqc) - 2.0 * p
    dm = jnp.sqrt(jnp.maximum(d2, 0.0))

    a = jnp.where(yr == yc, dm, HIGH_CST)

    rs = [jnp.full((BR, 128), HIGH_CST, jnp.float32) for _ in range(K + 1)]
    for t in range(N // 128):
        v = a[:, t * 128:(t + 1) * 128]
        for j in range(K + 1):
            lo = jnp.minimum(rs[j], v)
            v = jnp.maximum(rs[j], v)
            rs[j] = lo
    cand = jnp.concatenate(rs, axis=1)

    remaining = jnp.full((BR, 1), float(K + 1), jnp.float32)
    anchor = jnp.full((BR, 1), HIGH_CST, jnp.float32)
    for _ in range(K + 1):
        mn = jnp.min(cand, axis=1, keepdims=True)
        anchor = jnp.where(remaining > 0.0, mn, anchor)
        eq = cand == mn
        cnt = jnp.sum(jnp.where(eq, 1.0, 0.0), axis=1, keepdims=True)
        remaining = remaining - cnt
        cand = jnp.where(eq, HIGH_CST, cand)

    m_f = jnp.sum(jnp.where(dm <= anchor, 1.0, 0.0), axis=1,
                  keepdims=True) - 1.0
    m = m_f.astype(jnp.int32)
    psim = _tbl_gather(m, tbl, BR)
    dig_sum = jnp.sum(psim, keepdims=True)

    @pl.when(i == 0)
    def _():
        acc_ref[...] = jnp.zeros_like(acc_ref)

    acc_ref[...] = acc_ref[...] + dig_sum

    @pl.when(i == NB - 1)
    def _():
        cls = lax.broadcasted_iota(jnp.int32, (N_CLASSES, 1), 0)
        counts = jnp.sum((cls == yc).astype(jnp.int32), axis=1,
                         keepdims=True)
        psin = _tbl_gather(counts, tbl, N_CLASSES)
        avg_nx = jnp.sum((counts.astype(jnp.float32) * (1.0 / N)) * psin,
                         keepdims=True)
        mi = base_ref[...] - avg_nx - acc_ref[...] * (1.0 / N)
        out_ref[...] = jnp.maximum(mi / jnp.log(2.0), 0.0)


def _mi_call(x, sq_row, sq_col, y_col, y_row, tbl, base):
    return pl.pallas_call(
        _mi_kernel,
        grid=(NB,),
        in_specs=[
            pl.BlockSpec((N, D), lambda i: (0, 0)),
            pl.BlockSpec((1, N), lambda i: (0, 0)),
            pl.BlockSpec((BR, 1), lambda i: (i, 0)),
            pl.BlockSpec((1, N), lambda i: (0, 0)),
            pl.BlockSpec((BR, 1), lambda i: (i, 0)),
            pl.BlockSpec((TBL_HI, TBL_LO), lambda i: (0, 0)),
            pl.BlockSpec((1, 1), lambda i: (0, 0)),
        ],
        out_specs=pl.BlockSpec((1, 1), lambda i: (0, 0)),
        out_shape=jax.ShapeDtypeStruct((1, 1), jnp.float32),
        scratch_shapes=[pltpu.VMEM((1, 1), jnp.float32)],
    )(x, sq_row, sq_col, y_col, y_row, tbl, base)


def kernel(X, y):
    y32 = y.astype(jnp.int32)
    sq = jnp.sum(X * X, axis=1)
    tbl = _digamma(jnp.arange(TBL_HI * TBL_LO, dtype=jnp.float32))
    tbl = jnp.where(jnp.isfinite(tbl), tbl, 0.0)
    tbl = tbl.reshape(TBL_HI, TBL_LO)
    base = (_digamma(jnp.float32(N)) + _digamma(jnp.float32(K))).reshape(1, 1)
    out = _mi_call(X, sq.reshape(1, N), sq.reshape(N, 1),
                   y32.reshape(1, N), y32.reshape(N, 1), tbl, base)
    return out.reshape(())


# speedup vs baseline: 40.5545x; 1.0562x over previous
import jax
import jax.numpy as jnp
from jax import lax
from jax.experimental import pallas as pl
from jax.experimental.pallas import tpu as pltpu
from jax.scipy.special import digamma as _digamma

N = 4096
D = 256
K = 5
N_CLASSES = 10
BR = 1024
NB = N // BR
HIGH_CST = 10000000.0
TBL_HI = 40
TBL_LO = 128
_EXACT = lax.Precision.HIGHEST


def _tbl_gather(idx, tbl2d, rows):
    hi = lax.shift_right_logical(idx, 7)
    lo = lax.bitwise_and(idx, 127)
    oh_hi = (lax.broadcasted_iota(jnp.int32, (rows, TBL_HI), 1) == hi)
    row = lax.dot_general(oh_hi.astype(jnp.float32), tbl2d,
                          (((1,), (0,)), ((), ())),
                          precision=_EXACT,
                          preferred_element_type=jnp.float32)
    oh_lo = (lax.broadcasted_iota(jnp.int32, (rows, TBL_LO), 1) == lo)
    return jnp.sum(jnp.where(oh_lo, row, 0.0), axis=1, keepdims=True)


def _mi_kernel(xall_ref, sqc_ref, sqr_ref, yc_ref, yr_ref, tbl_ref,
               base_ref, out_ref, acc_ref):
    i = pl.program_id(0)

    xall = xall_ref[...]
    xr = xall_ref[pl.ds(i * BR, BR), :]
    sqc = sqc_ref[...]
    sqr = sqr_ref[...]
    yc = yc_ref[...]
    yr = yr_ref[...]
    tbl = tbl_ref[...]

    p = lax.dot_general(xr, xall, (((1,), (1,)), ((), ())),
                        preferred_element_type=jnp.float32)
    d2 = (sqr + sqc) - 2.0 * p
    dm = jnp.sqrt(jnp.maximum(d2, 0.0))

    a = jnp.where(yr == yc, dm, HIGH_CST)

    rs = [jnp.full((BR, 128), HIGH_CST, jnp.float32) for _ in range(K + 1)]
    for t in range(N // 128):
        v = a[:, t * 128:(t + 1) * 128]
        for j in range(K + 1):
            lo = jnp.minimum(rs[j], v)
            v = jnp.maximum(rs[j], v)
            rs[j] = lo
    cand = jnp.concatenate(rs, axis=1)

    remaining = jnp.full((BR, 1), float(K + 1), jnp.float32)
    anchor = jnp.full((BR, 1), HIGH_CST, jnp.float32)
    for _ in range(K + 1):
        mn = jnp.min(cand, axis=1, keepdims=True)
        anchor = jnp.where(remaining > 0.0, mn, anchor)
        eq = cand == mn
        cnt = jnp.sum(jnp.where(eq, 1.0, 0.0), axis=1, keepdims=True)
        remaining = remaining - cnt
        cand = jnp.where(eq, HIGH_CST, cand)

    m_f = jnp.sum(jnp.where(dm <= anchor, 1.0, 0.0), axis=1,
                  keepdims=True) - 1.0
    m = m_f.astype(jnp.int32)
    psim = _tbl_gather(m, tbl, BR)
    dig_sum = jnp.sum(psim, keepdims=True)

    @pl.when(i == 0)
    def _():
        acc_ref[...] = jnp.zeros_like(acc_ref)

    acc_ref[...] = acc_ref[...] + dig_sum

    @pl.when(i == NB - 1)
    def _():
        cls = lax.broadcasted_iota(jnp.int32, (N_CLASSES, 1), 0)
        counts = jnp.sum((cls == yc).astype(jnp.int32), axis=1,
                         keepdims=True)
        psin = _tbl_gather(counts, tbl, N_CLASSES)
        avg_nx = jnp.sum((counts.astype(jnp.float32) * (1.0 / N)) * psin,
                         keepdims=True)
        mi = base_ref[...] - avg_nx - acc_ref[...] * (1.0 / N)
        out_ref[...] = jnp.maximum(mi / jnp.log(2.0), 0.0)


def _mi_call(x, sq_row, sq_col, y_col, y_row, tbl, base):
    return pl.pallas_call(
        _mi_kernel,
        grid=(NB,),
        in_specs=[
            pl.BlockSpec((N, D), lambda i: (0, 0)),
            pl.BlockSpec((1, N), lambda i: (0, 0)),
            pl.BlockSpec((BR, 1), lambda i: (i, 0)),
            pl.BlockSpec((1, N), lambda i: (0, 0)),
            pl.BlockSpec((BR, 1), lambda i: (i, 0)),
            pl.BlockSpec((TBL_HI, TBL_LO), lambda i: (0, 0)),
            pl.BlockSpec((1, 1), lambda i: (0, 0)),
        ],
        out_specs=pl.BlockSpec((1, 1), lambda i: (0, 0)),
        out_shape=jax.ShapeDtypeStruct((1, 1), jnp.float32),
        scratch_shapes=[pltpu.VMEM((1, 1), jnp.float32)],
    )(x, sq_row, sq_col, y_col, y_row, tbl, base)


def kernel(X, y):
    y32 = y.astype(jnp.int32)
    sq = jnp.sum(X * X, axis=1)
    tbl = _digamma(jnp.arange(TBL_HI * TBL_LO, dtype=jnp.float32))
    tbl = jnp.where(jnp.isfinite(tbl), tbl, 0.0)
    tbl = tbl.reshape(TBL_HI, TBL_LO)
    base = (_digamma(jnp.float32(N)) + _digamma(jnp.float32(K))).reshape(1, 1)
    out = _mi_call(X, sq.reshape(1, N), sq.reshape(N, 1),
                   y32.reshape(1, N), y32.reshape(N, 1), tbl, base)
    return out.reshape(())
